# Initial kernel scaffold; baseline (speedup 1.0000x reference)
import jax
import jax.numpy as jnp
from jax.experimental import pallas as pl


def kernel(inp_sess, mask_1, mask_inf, lengths, adj_items, item_emb, prob_emb, cls_W, gru_Wih, gru_Whh, gru_bih, gru_bhh, ln1_g, ln1_b, ln2_g, ln2_b, ln3_g, ln3_b, ln4_g, ln4_b, a1, a2):
    raise NotImplementedError("write your pallas kernel here")


# speedup vs baseline: 2.2445x; 2.2445x over previous
import functools

import jax
import jax.numpy as jnp
from jax import lax
from jax.experimental import pallas as pl
from jax.experimental.pallas import tpu as pltpu
from jax.experimental.pallas import tpu_sc as plsc

N_ITEMS = 10000
M = 16
D = 128
K = 8
R_ITER = 4
B = 1024
L = 50
NP = 10240
ZERO_ROW = N_ITEMS - 1

_SC_NC = 2
_SC_NS = 16
_SC_NW = _SC_NC * _SC_NS


def _ln(x, g, b):
    mu = jnp.mean(x, axis=-1, keepdims=True)
    var = jnp.mean((x - mu) ** 2, axis=-1, keepdims=True)
    return (x - mu) / jnp.sqrt(var + 1e-5) * g + b


def _rownorm(x):
    n = jnp.sqrt(jnp.sum(x * x, axis=-1, keepdims=True))
    return x / jnp.maximum(n, 1e-12)


def _sc_gather(table, idx, chunk):
    nidx = idx.shape[0]
    dv = table.shape[1]
    bpw = nidx // _SC_NW
    iters = bpw // chunk
    mesh = plsc.VectorSubcoreMesh(core_axis_name="c", subcore_axis_name="s")

    @functools.partial(
        pl.kernel,
        mesh=mesh,
        out_type=jax.ShapeDtypeStruct((nidx, dv), table.dtype),
        scratch_types=[
            pltpu.VMEM((chunk,), jnp.int32),
            pltpu.VMEM((chunk, dv), table.dtype),
            pltpu.SemaphoreType.DMA,
        ],
    )
    def gather_kernel(table_hbm, idx_hbm, out_hbm, idx_v, rows_v, sem):
        wid = lax.axis_index("s") * _SC_NC + lax.axis_index("c")
        base = wid * bpw

        @pl.loop(0, iters)
        def _(i):
            off = base + i * chunk
            pltpu.sync_copy(idx_hbm.at[pl.ds(off, chunk)], idx_v)
            pltpu.async_copy(table_hbm.at[idx_v], rows_v, sem).wait()
            pltpu.sync_copy(rows_v, out_hbm.at[pl.ds(off, chunk)])

    return gather_kernel(table, idx)


def _prep_call(xp, prob, cls_wt, ln3g, ln3b):
    rb = 1024
    grid = NP // rb

    def body(xp_ref, prob_ref, w_ref, g_ref, b_ref, xn_ref, p_ref):
        xn_ref[...] = _rownorm(xp_ref[...])
        p = jnp.dot(prob_ref[...], w_ref[...],
                    preferred_element_type=jnp.float32)
        p_ref[...] = _ln(p, g_ref[...], b_ref[...])

    return pl.pallas_call(
        body,
        grid=(grid,),
        in_specs=[
            pl.BlockSpec((rb, D), lambda i: (i, 0)),
            pl.BlockSpec((rb, K), lambda i: (i, 0)),
            pl.BlockSpec((K, D), lambda i: (0, 0)),
            pl.BlockSpec((1, D), lambda i: (0, 0)),
            pl.BlockSpec((1, D), lambda i: (0, 0)),
        ],
        out_specs=[
            pl.BlockSpec((rb, D), lambda i: (i, 0)),
            pl.BlockSpec((rb, D), lambda i: (i, 0)),
        ],
        out_shape=[
            jax.ShapeDtypeStruct((NP, D), jnp.float32),
            jax.ShapeDtypeStruct((NP, D), jnp.float32),
        ],
    )(xp, prob, cls_wt, ln3g, ln3b)


def _routing_iters(z, xn):
    u = jnp.mean(z, axis=1) + xn
    for it in range(1, R_ITER + 1):
        if it < R_ITER:
            n2 = jnp.sum(u * u, axis=-1, keepdims=True)
            u = (n2 / (n2 + 1.0)) * u / jnp.maximum(jnp.sqrt(n2), 1e-12)
        if it == R_ITER:
            break
        p = jnp.sum(z * u[:, None, :], axis=2)
        p = p - jnp.max(p, axis=1, keepdims=True)
        e = jnp.exp(p)
        p = e / jnp.sum(e, axis=1, keepdims=True)
        u = jnp.sum(z * p[:, :, None], axis=1) + xn
    return u


def _route_a_call(z, xn):
    rb = 512
    grid = NP // rb

    def body(z_ref, xn_ref, u_ref, xn1_ref):
        u = _routing_iters(z_ref[...], xn_ref[...])
        u_ref[...] = u
        xn1_ref[...] = _rownorm(u)

    return pl.pallas_call(
        body,
        grid=(grid,),
        in_specs=[
            pl.BlockSpec((rb, M, D), lambda i: (i, 0, 0)),
            pl.BlockSpec((rb, D), lambda i: (i, 0)),
        ],
        out_specs=[
            pl.BlockSpec((rb, D), lambda i: (i, 0)),
            pl.BlockSpec((rb, D), lambda i: (i, 0)),
        ],
        out_shape=[
            jax.ShapeDtypeStruct((NP, D), jnp.float32),
            jax.ShapeDtypeStruct((NP, D), jnp.float32),
        ],
    )(z, xn)


def _route_b_call(z, xn1, xp, u1, ln1g, ln1b):
    rb = 512
    grid = NP // rb

    def body(z_ref, xn_ref, xp_ref, u1_ref, g_ref, b_ref, out_ref):
        u2 = _routing_iters(z_ref[...], xn_ref[...])
        iv = _ln(xp_ref[...] + u1_ref[...] + u2, g_ref[...], b_ref[...])
        row = pl.program_id(0) * rb + lax.broadcasted_iota(
            jnp.int32, (rb, D), 0)
        out_ref[...] = jnp.where(row >= ZERO_ROW, 0.0, iv)

    return pl.pallas_call(
        body,
        grid=(grid,),
        in_specs=[
            pl.BlockSpec((rb, M, D), lambda i: (i, 0, 0)),
            pl.BlockSpec((rb, D), lambda i: (i, 0)),
            pl.BlockSpec((rb, D), lambda i: (i, 0)),
            pl.BlockSpec((rb, D), lambda i: (i, 0)),
            pl.BlockSpec((1, D), lambda i: (0, 0)),
            pl.BlockSpec((1, D), lambda i: (0, 0)),
        ],
        out_specs=pl.BlockSpec((rb, D), lambda i: (i, 0)),
        out_shape=jax.ShapeDtypeStruct((NP, D), jnp.float32),
    )(z, xn1, xp, u1, ln1g, ln1b)


def _gru_call(gi, gc, wih_t, whh_t, bih, bhh, idx, ln2g, ln2b, ln4g, ln4b):
    nb = 256
    grid = B // nb

    def body(gi_ref, gc_ref, wi_ref, wh_ref, bi_ref, bh_ref, idx_ref,
             g2_ref, b2_ref, g4_ref, b4_ref, out_ref):
        wi = wi_ref[...]
        wh = wh_ref[...]
        bi = bi_ref[...]
        bh = bh_ref[...]
        sel = idx_ref[...]

        def step(t, carry):
            h, ht = carry
            x = jnp.concatenate([gi_ref[t], gc_ref[t]], axis=0)
            gates_i = jnp.dot(x, wi, preferred_element_type=jnp.float32) + bi
            gates_h = jnp.dot(h, wh, preferred_element_type=jnp.float32) + bh
            r = jax.nn.sigmoid(gates_i[:, :D] + gates_h[:, :D])
            zz = jax.nn.sigmoid(gates_i[:, D:2 * D] + gates_h[:, D:2 * D])
            nt = jnp.tanh(gates_i[:, 2 * D:] + r * gates_h[:, 2 * D:])
            h = (1.0 - zz) * nt + zz * h
            mask = jnp.concatenate([sel, sel], axis=0) == t
            ht = jnp.where(mask, h, ht)
            return h, ht

        h0 = jnp.zeros((2 * nb, D), jnp.float32)
        _, ht = lax.fori_loop(0, L, step, (h0, h0))
        out_ref[0] = _ln(ht[:nb], g2_ref[...], b2_ref[...])
        out_ref[1] = _ln(ht[nb:], g4_ref[...], b4_ref[...])

    return pl.pallas_call(
        body,
        grid=(grid,),
        in_specs=[
            pl.BlockSpec((L, nb, D), lambda b: (0, b, 0)),
            pl.BlockSpec((L, nb, D), lambda b: (0, b, 0)),
            pl.BlockSpec((D, 3 * D), lambda b: (0, 0)),
            pl.BlockSpec((D, 3 * D), lambda b: (0, 0)),
            pl.BlockSpec((1, 3 * D), lambda b: (0, 0)),
            pl.BlockSpec((1, 3 * D), lambda b: (0, 0)),
            pl.BlockSpec((nb, 1), lambda b: (b, 0)),
            pl.BlockSpec((1, D), lambda b: (0, 0)),
            pl.BlockSpec((1, D), lambda b: (0, 0)),
            pl.BlockSpec((1, D), lambda b: (0, 0)),
            pl.BlockSpec((1, D), lambda b: (0, 0)),
        ],
        out_specs=pl.BlockSpec((2, nb, D), lambda b: (0, b, 0)),
        out_shape=jax.ShapeDtypeStruct((2, B, D), jnp.float32),
    )(gi, gc, wih_t, whh_t, bih, bhh, idx, ln2g, ln2b, ln4g, ln4b)


def _scores_call(ht, ivt, pt, w12):
    cb = 512
    nc = N_ITEMS - 1
    grid = pl.cdiv(nc, cb)

    def body(ht_ref, iv_ref, p_ref, w_ref, s_ref, s1_ref, s2_ref):
        dn = (((1,), (1,)), ((), ()))
        s1 = lax.dot_general(ht_ref[0], iv_ref[...], dn,
                             preferred_element_type=jnp.float32)
        s2 = lax.dot_general(ht_ref[1], p_ref[...], dn,
                             preferred_element_type=jnp.float32)
        s1_ref[...] = s1
        s2_ref[...] = s2
        s_ref[...] = w_ref[0] * s1 + w_ref[1] * s2

    return pl.pallas_call(
        body,
        grid=(grid,),
        in_specs=[
            pl.BlockSpec((2, B, D), lambda j: (0, 0, 0)),
            pl.BlockSpec((cb, D), lambda j: (j, 0)),
            pl.BlockSpec((cb, D), lambda j: (j, 0)),
            pl.BlockSpec(memory_space=pltpu.SMEM),
        ],
        out_specs=[
            pl.BlockSpec((B, cb), lambda j: (0, j)),
            pl.BlockSpec((B, cb), lambda j: (0, j)),
            pl.BlockSpec((B, cb), lambda j: (0, j)),
        ],
        out_shape=[
            jax.ShapeDtypeStruct((B, nc), jnp.float32),
            jax.ShapeDtypeStruct((B, nc), jnp.float32),
            jax.ShapeDtypeStruct((B, nc), jnp.float32),
        ],
    )(ht, ivt, pt, w12)


def kernel(inp_sess, mask_1, mask_inf, lengths, adj_items, item_emb, prob_emb,
           cls_W, gru_Wih, gru_Whh, gru_bih, gru_bhh, ln1_g, ln1_b, ln2_g,
           ln2_b, ln3_g, ln3_b, ln4_g, ln4_b, a1, a2):
    f32 = jnp.float32
    n1 = N_ITEMS - 1

    xp = jnp.zeros((NP, D), f32).at[:n1].set(item_emb[1:].astype(f32))
    prob = (jnp.zeros((NP, K), f32)
            .at[:n1].set(prob_emb[1:].astype(f32))
            .at[ZERO_ROW].set(prob_emb[0].astype(f32)))
    adj_flat = (jnp.zeros((NP, M), jnp.int32)
                .at[:n1].set(adj_items[1:].astype(jnp.int32) - 1)
                ).reshape(-1)
    sess = jnp.where(inp_sess == 0, ZERO_ROW, inp_sess - 1).astype(jnp.int32)
    sess_flat = sess.T.reshape(-1)
    idx = ((lengths - 1) % L).astype(jnp.int32).reshape(B, 1)
    cls_wt = cls_W.T.astype(f32)
    wih_t = gru_Wih.T.astype(f32)
    whh_t = gru_Whh.T.astype(f32)
    bih = gru_bih.reshape(1, 3 * D).astype(f32)
    bhh = gru_bhh.reshape(1, 3 * D).astype(f32)
    r1 = lambda v: v.reshape(1, D).astype(f32)
    w12 = jax.nn.sigmoid(jnp.concatenate([a1, a2]).astype(f32))

    xn0, p_tab = _prep_call(xp, prob, cls_wt, r1(ln3_g), r1(ln3_b))
    z0 = _sc_gather(xn0, adj_flat, 512).reshape(NP, M, D)
    u1, xn1 = _route_a_call(z0, xn0)
    z1 = _sc_gather(xn1, adj_flat, 512).reshape(NP, M, D)
    ivt = _route_b_call(z1, xn1, xp, u1, r1(ln1_g), r1(ln1_b))

    gi = _sc_gather(ivt, sess_flat, 400).reshape(L, B, D)
    gc = _sc_gather(p_tab, sess_flat, 400).reshape(L, B, D)
    ht = _gru_call(gi, gc, wih_t, whh_t, bih, bhh, idx,
                   r1(ln2_g), r1(ln2_b), r1(ln4_g), r1(ln4_b))

    scores, scores1, scores2 = _scores_call(ht, ivt, p_tab, w12)
    return (scores, scores1, scores2)


# speedup vs baseline: 2.3176x; 1.0326x over previous
import functools

import jax
import jax.numpy as jnp
from jax import lax
from jax.experimental import pallas as pl
from jax.experimental.pallas import tpu as pltpu
from jax.experimental.pallas import tpu_sc as plsc

N_ITEMS = 10000
M = 16
D = 128
K = 8
R_ITER = 4
B = 1024
L = 50
NP = 10240
ZERO_ROW = N_ITEMS - 1

_SC_NC = 2
_SC_NS = 16
_SC_NW = _SC_NC * _SC_NS


def _ln(x, g, b):
    mu = jnp.mean(x, axis=-1, keepdims=True)
    var = jnp.mean((x - mu) ** 2, axis=-1, keepdims=True)
    return (x - mu) / jnp.sqrt(var + 1e-5) * g + b


def _rownorm(x):
    n = jnp.sqrt(jnp.sum(x * x, axis=-1, keepdims=True))
    return x / jnp.maximum(n, 1e-12)


def _sc_gather(table, idx, chunk):
    nidx = idx.shape[0]
    dv = table.shape[1]
    bpw = nidx // _SC_NW
    iters = bpw // chunk
    assert iters % 2 == 0 and iters * chunk == bpw and bpw * _SC_NW == nidx
    mesh = plsc.VectorSubcoreMesh(core_axis_name="c", subcore_axis_name="s")

    @functools.partial(
        pl.kernel,
        mesh=mesh,
        out_type=jax.ShapeDtypeStruct((nidx, dv), table.dtype),
        scratch_types=[
            pltpu.VMEM((bpw,), jnp.int32),
            pltpu.VMEM((chunk, dv), table.dtype),
            pltpu.VMEM((chunk, dv), table.dtype),
            pltpu.SemaphoreType.DMA,
            pltpu.SemaphoreType.DMA,
            pltpu.SemaphoreType.DMA,
            pltpu.SemaphoreType.DMA,
        ],
    )
    def gather_kernel(table_hbm, idx_hbm, out_hbm, idx_v, r0, r1,
                      sg0, sg1, sw0, sw1):
        wid = lax.axis_index("s") * _SC_NC + lax.axis_index("c")
        base = wid * bpw
        pltpu.sync_copy(idx_hbm.at[pl.ds(base, bpw)], idx_v)

        def drain(sem, buf):
            pltpu.make_async_copy(out_hbm.at[pl.ds(0, chunk)], buf, sem).wait()

        @pl.loop(0, iters, step=2)
        def _(i):
            @pl.when(i > 0)
            def _():
                drain(sw0, r0)
            pltpu.async_copy(
                table_hbm.at[idx_v.at[pl.ds(i * chunk, chunk)]], r0, sg0)

            @pl.when(i > 0)
            def _():
                drain(sw1, r1)
            pltpu.async_copy(
                table_hbm.at[idx_v.at[pl.ds((i + 1) * chunk, chunk)]], r1, sg1)

            drain(sg0, r0)
            pltpu.async_copy(r0, out_hbm.at[pl.ds(base + i * chunk, chunk)],
                             sw0)
            drain(sg1, r1)
            pltpu.async_copy(
                r1, out_hbm.at[pl.ds(base + (i + 1) * chunk, chunk)], sw1)

        drain(sw0, r0)
        drain(sw1, r1)

    return gather_kernel(table, idx)


def _prep_call(xp, prob, cls_wt, ln3g, ln3b):
    rb = 1024
    grid = NP // rb

    def body(xp_ref, prob_ref, w_ref, g_ref, b_ref, xn_ref, p_ref):
        xn_ref[...] = _rownorm(xp_ref[...])
        p = jnp.dot(prob_ref[...], w_ref[...],
                    preferred_element_type=jnp.float32)
        p_ref[...] = _ln(p, g_ref[...], b_ref[...])

    blk = lambda r, c: pl.BlockSpec((r, c), lambda i: (i, 0))
    full = lambda r, c: pl.BlockSpec((r, c), lambda i: (0, 0))
    return pl.pallas_call(
        body,
        grid=(grid,),
        in_specs=[blk(rb, D), blk(rb, K), full(K, D), full(1, D), full(1, D)],
        out_specs=[blk(rb, D)] * 2,
        out_shape=[
            jax.ShapeDtypeStruct((NP, D), jnp.float32),
            jax.ShapeDtypeStruct((NP, D), jnp.float32),
        ],
    )(xp, prob, cls_wt, ln3g, ln3b)


def _routing_iters(z, xn):
    u = jnp.mean(z, axis=1) + xn
    for it in range(1, R_ITER + 1):
        if it < R_ITER:
            n2 = jnp.sum(u * u, axis=-1, keepdims=True)
            u = (n2 / (n2 + 1.0)) * u / jnp.maximum(jnp.sqrt(n2), 1e-12)
        if it == R_ITER:
            break
        p = jnp.sum(z * u[:, None, :], axis=2)
        p = p - jnp.max(p, axis=1, keepdims=True)
        e = jnp.exp(p)
        p = e / jnp.sum(e, axis=1, keepdims=True)
        u = jnp.sum(z * p[:, :, None], axis=1) + xn
    return u


def _route_a_call(z, xn):
    rb = 512
    grid = NP // rb

    def body(z_ref, xn_ref, u_ref, xn1_ref):
        u = _routing_iters(z_ref[...], xn_ref[...])
        u_ref[...] = u
        xn1_ref[...] = _rownorm(u)

    return pl.pallas_call(
        body,
        grid=(grid,),
        in_specs=[
            pl.BlockSpec((rb, M, D), lambda i: (i, 0, 0)),
            pl.BlockSpec((rb, D), lambda i: (i, 0)),
        ],
        out_specs=[pl.BlockSpec((rb, D), lambda i: (i, 0))] * 2,
        out_shape=[
            jax.ShapeDtypeStruct((NP, D), jnp.float32),
            jax.ShapeDtypeStruct((NP, D), jnp.float32),
        ],
    )(z, xn)


def _route_b_call(z, xn1, xp, u1, ln1g, ln1b):
    rb = 512
    grid = NP // rb

    def body(z_ref, xn_ref, xp_ref, u1_ref, g_ref, b_ref, out_ref):
        u2 = _routing_iters(z_ref[...], xn_ref[...])
        iv = _ln(xp_ref[...] + u1_ref[...] + u2, g_ref[...], b_ref[...])
        row = pl.program_id(0) * rb + lax.broadcasted_iota(
            jnp.int32, (rb, D), 0)
        out_ref[...] = jnp.where(row >= ZERO_ROW, 0.0, iv)

    return pl.pallas_call(
        body,
        grid=(grid,),
        in_specs=[
            pl.BlockSpec((rb, M, D), lambda i: (i, 0, 0)),
            pl.BlockSpec((rb, D), lambda i: (i, 0)),
            pl.BlockSpec((rb, D), lambda i: (i, 0)),
            pl.BlockSpec((rb, D), lambda i: (i, 0)),
            pl.BlockSpec((1, D), lambda i: (0, 0)),
            pl.BlockSpec((1, D), lambda i: (0, 0)),
        ],
        out_specs=pl.BlockSpec((rb, D), lambda i: (i, 0)),
        out_shape=jax.ShapeDtypeStruct((NP, D), jnp.float32),
    )(z, xn1, xp, u1, ln1g, ln1b)


def _gru_call(gi, gc, wih_t, whh_t, bih, bhh, idx, ln2g, ln2b, ln4g, ln4b):
    nb = 256
    grid = B // nb

    def body(gi_ref, gc_ref, wi_ref, wh_ref, bi_ref, bh_ref, idx_ref,
             g2_ref, b2_ref, g4_ref, b4_ref, out_ref):
        wi = wi_ref[...]
        wh = wh_ref[...]
        bi = bi_ref[...]
        bh = bh_ref[...]
        sel = idx_ref[...]

        def step(t, carry):
            h, ht = carry
            x = jnp.concatenate([gi_ref[t], gc_ref[t]],
                                axis=0).astype(jnp.bfloat16)
            gates_i = jnp.dot(x, wi, preferred_element_type=jnp.float32) + bi
            gates_h = jnp.dot(h.astype(jnp.bfloat16), wh,
                              preferred_element_type=jnp.float32) + bh
            r = jax.nn.sigmoid(gates_i[:, :D] + gates_h[:, :D])
            zz = jax.nn.sigmoid(gates_i[:, D:2 * D] + gates_h[:, D:2 * D])
            nt = jnp.tanh(gates_i[:, 2 * D:] + r * gates_h[:, 2 * D:])
            h = (1.0 - zz) * nt + zz * h
            mask = jnp.concatenate([sel, sel], axis=0) == t
            ht = jnp.where(mask, h, ht)
            return h, ht

        h0 = jnp.zeros((2 * nb, D), jnp.float32)
        _, ht = lax.fori_loop(0, L, step, (h0, h0))
        out_ref[0] = _ln(ht[:nb], g2_ref[...], b2_ref[...])
        out_ref[1] = _ln(ht[nb:], g4_ref[...], b4_ref[...])

    return pl.pallas_call(
        body,
        grid=(grid,),
        in_specs=[
            pl.BlockSpec((L, nb, D), lambda b: (0, b, 0)),
            pl.BlockSpec((L, nb, D), lambda b: (0, b, 0)),
            pl.BlockSpec((D, 3 * D), lambda b: (0, 0)),
            pl.BlockSpec((D, 3 * D), lambda b: (0, 0)),
            pl.BlockSpec((1, 3 * D), lambda b: (0, 0)),
            pl.BlockSpec((1, 3 * D), lambda b: (0, 0)),
            pl.BlockSpec((nb, 1), lambda b: (b, 0)),
            pl.BlockSpec((1, D), lambda b: (0, 0)),
            pl.BlockSpec((1, D), lambda b: (0, 0)),
            pl.BlockSpec((1, D), lambda b: (0, 0)),
            pl.BlockSpec((1, D), lambda b: (0, 0)),
        ],
        out_specs=pl.BlockSpec((2, nb, D), lambda b: (0, b, 0)),
        out_shape=jax.ShapeDtypeStruct((2, B, D), jnp.float32),
    )(gi, gc, wih_t, whh_t, bih, bhh, idx, ln2g, ln2b, ln4g, ln4b)


def _scores_call(ht, ivt, pt, w12):
    cb = 512
    nc = N_ITEMS - 1
    grid = pl.cdiv(nc, cb)

    def body(ht_ref, iv_ref, p_ref, w_ref, s_ref, s1_ref, s2_ref):
        dn = (((1,), (1,)), ((), ()))
        s1 = lax.dot_general(ht_ref[0], iv_ref[...], dn,
                             preferred_element_type=jnp.float32)
        s2 = lax.dot_general(ht_ref[1], p_ref[...], dn,
                             preferred_element_type=jnp.float32)
        s1_ref[...] = s1
        s2_ref[...] = s2
        s_ref[...] = w_ref[0] * s1 + w_ref[1] * s2

    return pl.pallas_call(
        body,
        grid=(grid,),
        in_specs=[
            pl.BlockSpec((2, B, D), lambda j: (0, 0, 0)),
            pl.BlockSpec((cb, D), lambda j: (j, 0)),
            pl.BlockSpec((cb, D), lambda j: (j, 0)),
            pl.BlockSpec(memory_space=pltpu.SMEM),
        ],
        out_specs=[
            pl.BlockSpec((B, cb), lambda j: (0, j)),
            pl.BlockSpec((B, cb), lambda j: (0, j)),
            pl.BlockSpec((B, cb), lambda j: (0, j)),
        ],
        out_shape=[
            jax.ShapeDtypeStruct((B, nc), jnp.float32),
            jax.ShapeDtypeStruct((B, nc), jnp.float32),
            jax.ShapeDtypeStruct((B, nc), jnp.float32),
        ],
    )(ht, ivt, pt, w12)


def kernel(inp_sess, mask_1, mask_inf, lengths, adj_items, item_emb, prob_emb,
           cls_W, gru_Wih, gru_Whh, gru_bih, gru_bhh, ln1_g, ln1_b, ln2_g,
           ln2_b, ln3_g, ln3_b, ln4_g, ln4_b, a1, a2):
    f32 = jnp.float32
    bf16 = jnp.bfloat16
    n1 = N_ITEMS - 1

    xp = jnp.zeros((NP, D), f32).at[:n1].set(item_emb[1:].astype(f32))
    prob = (jnp.zeros((NP, K), f32)
            .at[:n1].set(prob_emb[1:].astype(f32))
            .at[ZERO_ROW].set(prob_emb[0].astype(f32)))
    adj_flat = (jnp.zeros((NP, M), jnp.int32)
                .at[:n1].set(adj_items[1:].astype(jnp.int32) - 1)
                ).reshape(-1)
    sess = jnp.where(inp_sess == 0, ZERO_ROW, inp_sess - 1).astype(jnp.int32)
    sess_flat = sess.T.reshape(-1)
    idx = ((lengths - 1) % L).astype(jnp.int32).reshape(B, 1)
    cls_wt = cls_W.T.astype(f32)
    wih_t = gru_Wih.T.astype(bf16)
    whh_t = gru_Whh.T.astype(bf16)
    bih = gru_bih.reshape(1, 3 * D).astype(f32)
    bhh = gru_bhh.reshape(1, 3 * D).astype(f32)
    r1 = lambda v: v.reshape(1, D).astype(f32)
    w12 = jax.nn.sigmoid(jnp.concatenate([a1, a2]).astype(f32))

    xn0, p_tab = _prep_call(xp, prob, cls_wt, r1(ln3_g), r1(ln3_b))
    z0 = _sc_gather(xn0, adj_flat, 320).reshape(NP, M, D)
    u1, xn1 = _route_a_call(z0, xn0)
    z1 = _sc_gather(xn1, adj_flat, 320).reshape(NP, M, D)
    ivt = _route_b_call(z1, xn1, xp, u1, r1(ln1_g), r1(ln1_b))

    gi = _sc_gather(ivt, sess_flat, 400).reshape(L, B, D)
    gc = _sc_gather(p_tab, sess_flat, 400).reshape(L, B, D)
    ht = _gru_call(gi, gc, wih_t, whh_t, bih, bhh, idx,
                   r1(ln2_g), r1(ln2_b), r1(ln4_g), r1(ln4_b))

    scores, scores1, scores2 = _scores_call(ht, ivt, p_tab, w12)
    return (scores, scores1, scores2)


# speedup vs baseline: 2.4758x; 1.0682x over previous
import functools

import jax
import jax.numpy as jnp
from jax import lax
from jax.experimental import pallas as pl
from jax.experimental.pallas import tpu as pltpu
from jax.experimental.pallas import tpu_sc as plsc

N_ITEMS = 10000
M = 16
D = 128
K = 8
R_ITER = 4
B = 1024
L = 50
NP = 10240
HP = NP // 2
ZERO_ROW = N_ITEMS - 1

_SC_NC = 2
_SC_NS = 16
_SC_NW = _SC_NC * _SC_NS


def _ln(x, g, b):
    mu = jnp.mean(x, axis=-1, keepdims=True)
    var = jnp.mean((x - mu) ** 2, axis=-1, keepdims=True)
    return (x - mu) / jnp.sqrt(var + 1e-5) * g + b


def _rownorm(x):
    n = jnp.sqrt(jnp.sum(x * x, axis=-1, keepdims=True))
    return x / jnp.maximum(n, 1e-12)


def _sc_gather(table, idx, chunk):
    nidx = idx.shape[0]
    dv = table.shape[1]
    bpw = nidx // _SC_NW
    iters = bpw // chunk
    assert iters % 2 == 0 and iters * chunk == bpw and bpw * _SC_NW == nidx
    mesh = plsc.VectorSubcoreMesh(core_axis_name="c", subcore_axis_name="s")

    @functools.partial(
        pl.kernel,
        mesh=mesh,
        out_type=jax.ShapeDtypeStruct((nidx, dv), table.dtype),
        scratch_types=[
            pltpu.VMEM((bpw,), jnp.int32),
            pltpu.VMEM((chunk, dv), table.dtype),
            pltpu.VMEM((chunk, dv), table.dtype),
            pltpu.SemaphoreType.DMA,
            pltpu.SemaphoreType.DMA,
            pltpu.SemaphoreType.DMA,
            pltpu.SemaphoreType.DMA,
        ],
    )
    def gather_kernel(table_hbm, idx_hbm, out_hbm, idx_v, r0, r1,
                      sg0, sg1, sw0, sw1):
        wid = lax.axis_index("s") * _SC_NC + lax.axis_index("c")
        base = wid * bpw
        pltpu.sync_copy(idx_hbm.at[pl.ds(base, bpw)], idx_v)

        def drain(sem, buf):
            pltpu.make_async_copy(out_hbm.at[pl.ds(0, chunk)], buf, sem).wait()

        @pl.loop(0, iters, step=2)
        def _(i):
            @pl.when(i > 0)
            def _():
                drain(sw0, r0)
            pltpu.async_copy(
                table_hbm.at[idx_v.at[pl.ds(i * chunk, chunk)]], r0, sg0)

            @pl.when(i > 0)
            def _():
                drain(sw1, r1)
            pltpu.async_copy(
                table_hbm.at[idx_v.at[pl.ds((i + 1) * chunk, chunk)]], r1, sg1)

            drain(sg0, r0)
            pltpu.async_copy(r0, out_hbm.at[pl.ds(base + i * chunk, chunk)],
                             sw0)
            drain(sg1, r1)
            pltpu.async_copy(
                r1, out_hbm.at[pl.ds(base + (i + 1) * chunk, chunk)], sw1)

        drain(sw0, r0)
        drain(sw1, r1)

    return gather_kernel(table, idx)


def _prep_call(xp, prob, cls_wt, ln3g, ln3b):
    rb = 1024
    grid = NP // rb

    def body(xp_ref, prob_ref, w_ref, g_ref, b_ref, xn_ref, p_ref):
        xn_ref[...] = _rownorm(xp_ref[...])
        p = jnp.dot(prob_ref[...], w_ref[...],
                    preferred_element_type=jnp.float32)
        p_ref[...] = _ln(p, g_ref[...], b_ref[...])

    blk = lambda r, c: pl.BlockSpec((r, c), lambda i: (i, 0))
    full = lambda r, c: pl.BlockSpec((r, c), lambda i: (0, 0))
    return pl.pallas_call(
        body,
        grid=(grid,),
        in_specs=[blk(rb, D), blk(rb, K), full(K, D), full(1, D), full(1, D)],
        out_specs=[blk(rb, D)] * 2,
        out_shape=[
            jax.ShapeDtypeStruct((NP, D), jnp.float32),
            jax.ShapeDtypeStruct((NP, D), jnp.float32),
        ],
    )(xp, prob, cls_wt, ln3g, ln3b)


def _routing_iters(z, xn):
    u = jnp.mean(z, axis=1) + xn
    for it in range(1, R_ITER + 1):
        if it < R_ITER:
            n2 = jnp.sum(u * u, axis=-1, keepdims=True)
            u = (n2 / (n2 + 1.0)) * u / jnp.maximum(jnp.sqrt(n2), 1e-12)
        if it == R_ITER:
            break
        p = jnp.sum(z * u[:, None, :], axis=2)
        p = p - jnp.max(p, axis=1, keepdims=True)
        e = jnp.exp(p)
        p = e / jnp.sum(e, axis=1, keepdims=True)
        u = jnp.sum(z * p[:, :, None], axis=1) + xn
    return u


def _route_a_part(z, xn, part):
    rb = 512
    grid = HP // rb
    off = part * grid

    def body(z_ref, xn_ref, u_ref, xn1_ref):
        u = _routing_iters(z_ref[...], xn_ref[...])
        u_ref[...] = u
        xn1_ref[...] = _rownorm(u)

    return pl.pallas_call(
        body,
        grid=(grid,),
        in_specs=[
            pl.BlockSpec((rb, M, D), lambda i: (i, 0, 0)),
            pl.BlockSpec((rb, D), lambda i: (i + off, 0)),
        ],
        out_specs=[pl.BlockSpec((rb, D), lambda i: (i, 0))] * 2,
        out_shape=[
            jax.ShapeDtypeStruct((HP, D), jnp.float32),
            jax.ShapeDtypeStruct((HP, D), jnp.float32),
        ],
    )(z, xn)


def _route_b_part(z, xn1, xp, u1, ln1g, ln1b, part):
    rb = 512
    grid = HP // rb
    off = part * grid

    def body(z_ref, xn_ref, xp_ref, u1_ref, g_ref, b_ref, out_ref):
        u2 = _routing_iters(z_ref[...], xn_ref[...])
        iv = _ln(xp_ref[...] + u1_ref[...] + u2, g_ref[...], b_ref[...])
        row = (pl.program_id(0) + off) * rb + lax.broadcasted_iota(
            jnp.int32, (rb, D), 0)
        out_ref[...] = jnp.where(row >= ZERO_ROW, 0.0, iv)

    return pl.pallas_call(
        body,
        grid=(grid,),
        in_specs=[
            pl.BlockSpec((rb, M, D), lambda i: (i, 0, 0)),
            pl.BlockSpec((rb, D), lambda i: (i + off, 0)),
            pl.BlockSpec((rb, D), lambda i: (i + off, 0)),
            pl.BlockSpec((rb, D), lambda i: (i + off, 0)),
            pl.BlockSpec((1, D), lambda i: (0, 0)),
            pl.BlockSpec((1, D), lambda i: (0, 0)),
        ],
        out_specs=pl.BlockSpec((rb, D), lambda i: (i, 0)),
        out_shape=jax.ShapeDtypeStruct((HP, D), jnp.float32),
    )(z, xn1, xp, u1, ln1g, ln1b)


def _gru_call(gi, gc, wih_t, whh_t, bih, bhh, idx, ln2g, ln2b, ln4g, ln4b):
    nb = 256
    grid = B // nb
    bf16 = jnp.bfloat16

    def body(gi_ref, gc_ref, wi_ref, wh_ref, bi_ref, bh_ref, idx_ref,
             g2_ref, b2_ref, g4_ref, b4_ref, out_ref, gia_ref):
        wh = wh_ref[...]
        bh = bh_ref[...]
        x_all = jnp.concatenate([gi_ref[...], gc_ref[...]],
                                axis=1)
        gia = jnp.dot(x_all.reshape(L * 2 * nb, D).astype(bf16), wi_ref[...],
                      preferred_element_type=jnp.float32) + bi_ref[...]
        gia_ref[...] = gia.reshape(L, 2 * nb, 3 * D).astype(bf16)
        sel = idx_ref[...]
        sel2 = jnp.concatenate([sel, sel], axis=0)

        def step(t, carry):
            h, ht = carry
            gi_t = gia_ref[t]
            gh = jnp.dot(h.astype(bf16), wh,
                         preferred_element_type=jnp.float32) + bh
            r = jax.nn.sigmoid(gi_t[:, :D] + gh[:, :D])
            zz = jax.nn.sigmoid(gi_t[:, D:2 * D] + gh[:, D:2 * D])
            nt = jnp.tanh(gi_t[:, 2 * D:] + r * gh[:, 2 * D:])
            h = (1.0 - zz) * nt + zz * h
            ht = jnp.where(sel2 == t, h, ht)
            return h, ht

        h0 = jnp.zeros((2 * nb, D), jnp.float32)
        _, ht = lax.fori_loop(0, L, step, (h0, h0))
        out_ref[0] = _ln(ht[:nb], g2_ref[...], b2_ref[...])
        out_ref[1] = _ln(ht[nb:], g4_ref[...], b4_ref[...])

    return pl.pallas_call(
        body,
        grid=(grid,),
        in_specs=[
            pl.BlockSpec((L, nb, D), lambda b: (0, b, 0)),
            pl.BlockSpec((L, nb, D), lambda b: (0, b, 0)),
            pl.BlockSpec((D, 3 * D), lambda b: (0, 0)),
            pl.BlockSpec((D, 3 * D), lambda b: (0, 0)),
            pl.BlockSpec((1, 3 * D), lambda b: (0, 0)),
            pl.BlockSpec((1, 3 * D), lambda b: (0, 0)),
            pl.BlockSpec((nb, 1), lambda b: (b, 0)),
            pl.BlockSpec((1, D), lambda b: (0, 0)),
            pl.BlockSpec((1, D), lambda b: (0, 0)),
            pl.BlockSpec((1, D), lambda b: (0, 0)),
            pl.BlockSpec((1, D), lambda b: (0, 0)),
        ],
        out_specs=pl.BlockSpec((2, nb, D), lambda b: (0, b, 0)),
        out_shape=jax.ShapeDtypeStruct((2, B, D), jnp.float32),
        scratch_shapes=[pltpu.VMEM((L, 2 * nb, 3 * D), bf16)],
    )(gi, gc, wih_t, whh_t, bih, bhh, idx, ln2g, ln2b, ln4g, ln4b)


def _scores_call(ht, ivt, pt, w12):
    cb = 512
    nc = N_ITEMS - 1
    grid = pl.cdiv(nc, cb)
    bf16 = jnp.bfloat16

    def body(ht_ref, iv_ref, p_ref, w_ref, s_ref, s1_ref, s2_ref):
        dn = (((1,), (1,)), ((), ()))
        s1 = lax.dot_general(ht_ref[0].astype(bf16),
                             iv_ref[...].astype(bf16), dn,
                             preferred_element_type=jnp.float32)
        s2 = lax.dot_general(ht_ref[1].astype(bf16),
                             p_ref[...].astype(bf16), dn,
                             preferred_element_type=jnp.float32)
        s1_ref[...] = s1
        s2_ref[...] = s2
        s_ref[...] = w_ref[0] * s1 + w_ref[1] * s2

    return pl.pallas_call(
        body,
        grid=(grid,),
        in_specs=[
            pl.BlockSpec((2, B, D), lambda j: (0, 0, 0)),
            pl.BlockSpec((cb, D), lambda j: (j, 0)),
            pl.BlockSpec((cb, D), lambda j: (j, 0)),
            pl.BlockSpec(memory_space=pltpu.SMEM),
        ],
        out_specs=[
            pl.BlockSpec((B, cb), lambda j: (0, j)),
            pl.BlockSpec((B, cb), lambda j: (0, j)),
            pl.BlockSpec((B, cb), lambda j: (0, j)),
        ],
        out_shape=[
            jax.ShapeDtypeStruct((B, nc), jnp.float32),
            jax.ShapeDtypeStruct((B, nc), jnp.float32),
            jax.ShapeDtypeStruct((B, nc), jnp.float32),
        ],
    )(ht, ivt, pt, w12)


def kernel(inp_sess, mask_1, mask_inf, lengths, adj_items, item_emb, prob_emb,
           cls_W, gru_Wih, gru_Whh, gru_bih, gru_bhh, ln1_g, ln1_b, ln2_g,
           ln2_b, ln3_g, ln3_b, ln4_g, ln4_b, a1, a2):
    f32 = jnp.float32
    bf16 = jnp.bfloat16
    n1 = N_ITEMS - 1

    xp = jnp.zeros((NP, D), f32).at[:n1].set(item_emb[1:].astype(f32))
    prob = (jnp.zeros((NP, K), f32)
            .at[:n1].set(prob_emb[1:].astype(f32))
            .at[ZERO_ROW].set(prob_emb[0].astype(f32)))
    adj_flat = (jnp.zeros((NP, M), jnp.int32)
                .at[:n1].set(adj_items[1:].astype(jnp.int32) - 1)
                ).reshape(-1)
    adj_a, adj_b = adj_flat[:HP * M], adj_flat[HP * M:]
    sess = jnp.where(inp_sess == 0, ZERO_ROW, inp_sess - 1).astype(jnp.int32)
    sess_flat = sess.T.reshape(-1)
    idx = ((lengths - 1) % L).astype(jnp.int32).reshape(B, 1)
    cls_wt = cls_W.T.astype(f32)
    wih_t = gru_Wih.T.astype(bf16)
    whh_t = gru_Whh.T.astype(bf16)
    bih = gru_bih.reshape(1, 3 * D).astype(f32)
    bhh = gru_bhh.reshape(1, 3 * D).astype(f32)
    r1 = lambda v: v.reshape(1, D).astype(f32)
    w12 = jax.nn.sigmoid(jnp.concatenate([a1, a2]).astype(f32))

    xn0, p_tab = _prep_call(xp, prob, cls_wt, r1(ln3_g), r1(ln3_b))
    z0a = _sc_gather(xn0, adj_a, 320).reshape(HP, M, D)
    z0b = _sc_gather(xn0, adj_b, 320).reshape(HP, M, D)
    u1a, xn1a = _route_a_part(z0a, xn0, 0)
    u1b, xn1b = _route_a_part(z0b, xn0, 1)
    u1 = jnp.concatenate([u1a, u1b], axis=0)
    xn1 = jnp.concatenate([xn1a, xn1b], axis=0)
    z1a = _sc_gather(xn1, adj_a, 320).reshape(HP, M, D)
    z1b = _sc_gather(xn1, adj_b, 320).reshape(HP, M, D)
    iva = _route_b_part(z1a, xn1, xp, u1, r1(ln1_g), r1(ln1_b), 0)
    ivb = _route_b_part(z1b, xn1, xp, u1, r1(ln1_g), r1(ln1_b), 1)
    ivt = jnp.concatenate([iva, ivb], axis=0)

    gi = _sc_gather(ivt, sess_flat, 400).reshape(L, B, D)
    gc = _sc_gather(p_tab, sess_flat, 400).reshape(L, B, D)
    ht = _gru_call(gi, gc, wih_t, whh_t, bih, bhh, idx,
                   r1(ln2_g), r1(ln2_b), r1(ln4_g), r1(ln4_b))

    scores, scores1, scores2 = _scores_call(ht, ivt, p_tab, w12)
    return (scores, scores1, scores2)
